# baseline (speedup 1.0000x reference)
import jax
import jax.numpy as jnp
from jax import lax
from jax.experimental import pallas as pl
from jax.experimental.pallas import tpu as pltpu
from jax.experimental.pallas import tpu_sc as plsc

BATCH = 16384
DIM = 64
NC = 2
NS = 16
NW = NC * NS
BPW = BATCH // NW
CH = 128
NCH = BPW // CH
L = 16


def _body(tw_hbm, cw_hbm, temb_hbm, cemb_hbm, out_hbm,
          idx_t, idx_c, rows_t, rows_c, out_v, sem):
    cid = lax.axis_index("c")
    sid = lax.axis_index("s")
    wid = sid * NC + cid
    base = wid * BPW

    for j in range(NCH):
        pltpu.sync_copy(tw_hbm.at[pl.ds(base + j * CH, CH)], idx_t.at[j])
        pltpu.sync_copy(cw_hbm.at[pl.ds(base + j * CH, CH)], idx_c.at[j])

    copies = []
    for j in range(NCH):
        copies.append(pltpu.async_copy(
            temb_hbm.at[idx_t.at[j]], rows_t.at[pl.ds(j * CH, CH)], sem))
        copies.append(pltpu.async_copy(
            cemb_hbm.at[idx_c.at[j]], rows_c.at[pl.ds(j * CH, CH)], sem))
    for cp in copies:
        cp.wait()

    lane = lax.iota(jnp.int32, L)

    def group(g, carry):
        row = g * L + lane
        acc = jnp.zeros((L,), jnp.float32)
        for d in range(DIM):
            col = jnp.full((L,), d, jnp.int32)
            tv = plsc.load_gather(rows_t, [row, col])
            cv = plsc.load_gather(rows_c, [row, col])
            acc = acc + tv * cv
        out_v[pl.ds(g * L, L)] = acc
        return carry

    lax.fori_loop(0, BPW // L, group, 0)

    pltpu.sync_copy(out_v, out_hbm.at[pl.ds(base, BPW)])


def kernel(target_word, context_word, target_emb, context_emb):
    tw = target_word.astype(jnp.int32)
    cw = context_word.astype(jnp.int32)
    mesh = plsc.VectorSubcoreMesh(
        core_axis_name="c", subcore_axis_name="s",
        num_cores=NC, num_subcores=NS)
    run = pl.kernel(
        _body,
        out_type=jax.ShapeDtypeStruct((BATCH,), jnp.float32),
        mesh=mesh,
        scratch_types=[
            pltpu.VMEM((NCH, CH), jnp.int32),
            pltpu.VMEM((NCH, CH), jnp.int32),
            pltpu.VMEM((BPW, DIM), jnp.float32),
            pltpu.VMEM((BPW, DIM), jnp.float32),
            pltpu.VMEM((BPW,), jnp.float32),
            pltpu.SemaphoreType.DMA,
        ],
        compiler_params=pltpu.CompilerParams(
            needs_layout_passes=False, use_tc_tiling_on_sc=False),
    )
    return run(tw, cw, target_emb, context_emb)


# speedup vs baseline: 2.2576x; 2.2576x over previous
import jax
import jax.numpy as jnp
from jax import lax
from jax.experimental import pallas as pl
from jax.experimental.pallas import tpu as pltpu
from jax.experimental.pallas import tpu_sc as plsc

BATCH = 16384
DIM = 64
ROWS_PER_TILE = 8
NTILE = 1000000 // ROWS_PER_TILE
NC = 2
NS = 16
NW = NC * NS
BPW = BATCH // NW
L = 16
CH = 16
NCHUNK = BPW // CH
IDXROW = 128
NIDXROW = BPW // IDXROW


def _body(tw_hbm, cw_hbm, temb_hbm, cemb_hbm, out_hbm,
          idx_t, idx_c, rt0, rt1, rc0, rc1, out_v,
          st0, st1, sc0, sc1):
    cid = lax.axis_index("c")
    sid = lax.axis_index("s")
    wid = sid * NC + cid
    base = wid * BPW

    for j in range(NIDXROW):
        pltpu.sync_copy(tw_hbm.at[pl.ds(base + j * IDXROW, IDXROW)], idx_t.at[j])
        pltpu.sync_copy(cw_hbm.at[pl.ds(base + j * IDXROW, IDXROW)], idx_c.at[j])

    lane = lax.iota(jnp.int32, L)
    zero = jnp.zeros((L,), jnp.int32)

    def fire(ch, rt, rc, sem_t, sem_c):
        j = ch >> 3
        col = (ch & 7) * L
        tiles_t = idx_t[j, pl.ds(col, L)] >> 3
        tiles_c = idx_c[j, pl.ds(col, L)] >> 3
        for i in range(CH):
            ti = jnp.sum(jnp.where(lane == i, tiles_t, zero))
            pltpu.make_async_copy(
                temb_hbm.at[pl.ds(ti, 1)], rt.at[pl.ds(i, 1)], sem_t).start()
            ci = jnp.sum(jnp.where(lane == i, tiles_c, zero))
            pltpu.make_async_copy(
                cemb_hbm.at[pl.ds(ci, 1)], rc.at[pl.ds(i, 1)], sem_c).start()

    def drain(rt, rc, sem_t, sem_c):
        for i in range(CH):
            pltpu.make_async_copy(
                temb_hbm.at[pl.ds(0, 1)], rt.at[pl.ds(i, 1)], sem_t).wait()
            pltpu.make_async_copy(
                cemb_hbm.at[pl.ds(0, 1)], rc.at[pl.ds(i, 1)], sem_c).wait()

    def compute(ch, rt, rc):
        j = ch >> 3
        col = (ch & 7) * L
        sub_t = idx_t[j, pl.ds(col, L)] & 7
        sub_c = idx_c[j, pl.ds(col, L)] & 7
        acc = jnp.zeros((L,), jnp.float32)
        for d in range(DIM):
            dv = jnp.full((L,), d, jnp.int32)
            tv = plsc.load_gather(rt, [lane, sub_t, dv])
            cv = plsc.load_gather(rc, [lane, sub_c, dv])
            acc = acc + tv * cv
        out_v[pl.ds(ch * CH, CH)] = acc

    fire(0, rt0, rc0, st0, sc0)

    def step(s, carry):
        ch0 = 2 * s
        ch1 = 2 * s + 1
        fire(ch1, rt1, rc1, st1, sc1)
        drain(rt0, rc0, st0, sc0)
        compute(ch0, rt0, rc0)
        @pl.when(ch0 + 2 < NCHUNK)
        def _():
            fire(ch0 + 2, rt0, rc0, st0, sc0)
        drain(rt1, rc1, st1, sc1)
        compute(ch1, rt1, rc1)
        return carry

    lax.fori_loop(0, NCHUNK // 2, step, 0)

    pltpu.sync_copy(out_v, out_hbm.at[pl.ds(base, BPW)])


def kernel(target_word, context_word, target_emb, context_emb):
    tw = target_word.astype(jnp.int32)
    cw = context_word.astype(jnp.int32)
    temb3 = target_emb.reshape(NTILE, ROWS_PER_TILE, DIM)
    cemb3 = context_emb.reshape(NTILE, ROWS_PER_TILE, DIM)
    mesh = plsc.VectorSubcoreMesh(
        core_axis_name="c", subcore_axis_name="s",
        num_cores=NC, num_subcores=NS)
    run = pl.kernel(
        _body,
        out_type=jax.ShapeDtypeStruct((BATCH,), jnp.float32),
        mesh=mesh,
        scratch_types=[
            pltpu.VMEM((NIDXROW, IDXROW), jnp.int32),
            pltpu.VMEM((NIDXROW, IDXROW), jnp.int32),
            pltpu.VMEM((CH, ROWS_PER_TILE, DIM), jnp.float32),
            pltpu.VMEM((CH, ROWS_PER_TILE, DIM), jnp.float32),
            pltpu.VMEM((CH, ROWS_PER_TILE, DIM), jnp.float32),
            pltpu.VMEM((CH, ROWS_PER_TILE, DIM), jnp.float32),
            pltpu.VMEM((BPW,), jnp.float32),
            pltpu.SemaphoreType.DMA,
            pltpu.SemaphoreType.DMA,
            pltpu.SemaphoreType.DMA,
            pltpu.SemaphoreType.DMA,
        ],
        compiler_params=pltpu.CompilerParams(
            needs_layout_passes=False, use_tc_tiling_on_sc=True),
    )
    return run(tw, cw, temb3, cemb3)
